# baseline (speedup 1.0000x reference)
import functools

import numpy as np
import jax
import jax.numpy as jnp
from jax import lax
from jax.experimental import pallas as pl
from jax.experimental.pallas import tpu as pltpu
from jax.experimental.pallas import tpu_sc as plsc

_D = 768
_MAX_LEN = 2048
_BATCH = 4
_N = _BATCH * _MAX_LEN
_SCALE = float(np.sqrt(np.float32(_D)))


def _positional_encoding() -> np.ndarray:
    pos = np.arange(_MAX_LEN)[:, np.newaxis].astype(np.float64)
    i = np.arange(_D)[np.newaxis, :].astype(np.float64)
    angle_rates = 1.0 / np.power(10000.0, 2.0 * (i // 2) / np.float32(_D))
    angle_rads = pos * angle_rates
    angle_rads[:, 0::2] = np.sin(angle_rads[:, 0::2])
    angle_rads[:, 1::2] = np.cos(angle_rads[:, 1::2])
    return angle_rads.astype(np.float32)


_POS_ENC = _positional_encoding()

_INFO = plsc.get_sparse_core_info()
_NC = _INFO.num_cores
_NS = _INFO.num_subcores
_NW = _NC * _NS
_B_PER_W = _N // _NW
_CHUNK = 64
_N_CHUNKS = _B_PER_W // _CHUNK
_LANES = 16
_COLS = _D // _LANES

_mesh = plsc.VectorSubcoreMesh(core_axis_name="c", subcore_axis_name="s")


@functools.partial(
    pl.kernel,
    mesh=_mesh,
    out_type=jax.ShapeDtypeStruct((_N, _D), jnp.float32),
    scratch_types=[
        pltpu.VMEM((_CHUNK,), jnp.int32),
        pltpu.VMEM((_CHUNK, _D), jnp.float32),
        pltpu.VMEM((_CHUNK, _D), jnp.float32),
        pltpu.SemaphoreType.DMA,
    ],
)
def _embed(x_hbm, table_hbm, pos_hbm, out_hbm, idx_v, rows_v, pos_v, sem):
    wid = lax.axis_index("s") * _NC + lax.axis_index("c")
    w_base = wid * _B_PER_W
    for ci in range(_N_CHUNKS):
        base = w_base + ci * _CHUNK
        t0 = lax.rem(base, _MAX_LEN)
        pltpu.sync_copy(x_hbm.at[pl.ds(base, _CHUNK)], idx_v)
        gather = pltpu.async_copy(table_hbm.at[idx_v], rows_v, sem)
        pltpu.sync_copy(pos_hbm.at[pl.ds(t0, _CHUNK)], pos_v)
        gather.wait()

        def _row(r, carry):
            for j in range(_COLS):
                sl = pl.ds(j * _LANES, _LANES)
                rows_v[r, sl] = rows_v[r, sl] * _SCALE + pos_v[r, sl]
            return carry

        lax.fori_loop(0, _CHUNK, _row, 0)
        pltpu.sync_copy(rows_v, out_hbm.at[pl.ds(base, _CHUNK)])


def kernel(x, table):
    xf = jnp.reshape(x, (_N,)).astype(jnp.int32)
    out = _embed(xf, table, jnp.asarray(_POS_ENC))
    return jnp.reshape(out, (_BATCH, _MAX_LEN, _D))


# speedup vs baseline: 1.0295x; 1.0295x over previous
import functools

import numpy as np
import jax
import jax.numpy as jnp
from jax import lax
from jax.experimental import pallas as pl
from jax.experimental.pallas import tpu as pltpu
from jax.experimental.pallas import tpu_sc as plsc

_D = 768
_MAX_LEN = 2048
_BATCH = 4
_N = _BATCH * _MAX_LEN
_SCALE = float(np.sqrt(np.float32(_D)))


def _positional_encoding() -> np.ndarray:
    pos = np.arange(_MAX_LEN)[:, np.newaxis].astype(np.float64)
    i = np.arange(_D)[np.newaxis, :].astype(np.float64)
    angle_rates = 1.0 / np.power(10000.0, 2.0 * (i // 2) / np.float32(_D))
    angle_rads = pos * angle_rates
    angle_rads[:, 0::2] = np.sin(angle_rads[:, 0::2])
    angle_rads[:, 1::2] = np.cos(angle_rads[:, 1::2])
    return angle_rads.astype(np.float32)


_POS_ENC = _positional_encoding()

_INFO = plsc.get_sparse_core_info()
_NC = _INFO.num_cores
_NS = _INFO.num_subcores
_NW = _NC * _NS
_T_PER_W = _MAX_LEN // _NW
_CHUNK = 32
_SPLIT = _T_PER_W // _CHUNK
_NCHUNKS = _BATCH * _SPLIT
_NBUF = 3
_LANES = 16
_COLS = _D // _LANES

_mesh = plsc.VectorSubcoreMesh(core_axis_name="c", subcore_axis_name="s")


@functools.partial(
    pl.kernel,
    mesh=_mesh,
    out_type=jax.ShapeDtypeStruct((_N, _D), jnp.float32),
    scratch_types=[
        pltpu.VMEM((_T_PER_W, _D), jnp.float32),
        *[pltpu.VMEM((_CHUNK,), jnp.int32) for _ in range(_NBUF)],
        *[pltpu.VMEM((_CHUNK, _D), jnp.float32) for _ in range(_NBUF)],
        *[pltpu.SemaphoreType.DMA for _ in range(2 * _NBUF)],
    ],
)
def _embed(x_hbm, table_hbm, pos_hbm, out_hbm,
           pos_v, i0, i1, i2, r0, r1, r2, g0, g1, g2, w0, w1, w2):
    idx_v = (i0, i1, i2)
    rows_v = (r0, r1, r2)
    gsem = (g0, g1, g2)
    wsem = (w0, w1, w2)
    wid = lax.axis_index("s") * _NC + lax.axis_index("c")
    t_base = wid * _T_PER_W

    pltpu.sync_copy(pos_hbm.at[pl.ds(t_base, _T_PER_W)], pos_v)

    def chunk_base(k):
        b, c = divmod(k, _SPLIT)
        return b * _MAX_LEN + t_base + c * _CHUNK

    def fire_gather(k):
        j = k % _NBUF
        pltpu.sync_copy(x_hbm.at[pl.ds(chunk_base(k), _CHUNK)], idx_v[j])
        return pltpu.async_copy(table_hbm.at[idx_v[j]], rows_v[j], gsem[j])

    gathers = {0: fire_gather(0)}
    writes = {}
    for k in range(_NCHUNKS):
        j = k % _NBUF
        if k + 1 < _NCHUNKS:
            if k + 1 - _NBUF in writes:
                writes[k + 1 - _NBUF].wait()
            gathers[k + 1] = fire_gather(k + 1)
        gathers[k].wait()

        t_local = (k % _SPLIT) * _CHUNK
        rv = rows_v[j]

        def _row(r, carry):
            for col in range(_COLS):
                sl = pl.ds(col * _LANES, _LANES)
                rv[r, sl] = rv[r, sl] * _SCALE + pos_v[t_local + r, sl]
            return carry

        lax.fori_loop(0, _CHUNK, _row, 0)
        writes[k] = pltpu.async_copy(
            rv, out_hbm.at[pl.ds(chunk_base(k), _CHUNK)], wsem[j])
    for k in range(_NCHUNKS - _NBUF, _NCHUNKS):
        writes[k].wait()


def kernel(x, table):
    xf = jnp.reshape(x, (_N,)).astype(jnp.int32)
    out = _embed(xf, table, jnp.asarray(_POS_ENC))
    return jnp.reshape(out, (_BATCH, _MAX_LEN, _D))


# speedup vs baseline: 1.2898x; 1.2528x over previous
import functools

import numpy as np
import jax
import jax.numpy as jnp
from jax import lax
from jax.experimental import pallas as pl
from jax.experimental.pallas import tpu as pltpu
from jax.experimental.pallas import tpu_sc as plsc

_D = 768
_MAX_LEN = 2048
_BATCH = 4
_N = _BATCH * _MAX_LEN
_SCALE = float(np.sqrt(np.float32(_D)))


def _positional_encoding() -> np.ndarray:
    pos = np.arange(_MAX_LEN)[:, np.newaxis].astype(np.float64)
    i = np.arange(_D)[np.newaxis, :].astype(np.float64)
    angle_rates = 1.0 / np.power(10000.0, 2.0 * (i // 2) / np.float32(_D))
    angle_rads = pos * angle_rates
    angle_rads[:, 0::2] = np.sin(angle_rads[:, 0::2])
    angle_rads[:, 1::2] = np.cos(angle_rads[:, 1::2])
    return angle_rads.astype(np.float32)


_POS_ENC = _positional_encoding()

_INFO = plsc.get_sparse_core_info()
_NC = _INFO.num_cores
_NS = _INFO.num_subcores
_NW = _NC * _NS
_T_PER_W = _MAX_LEN // _NW
_CHUNK = 32
_SPLIT = _T_PER_W // _CHUNK
_NCHUNKS = _BATCH * _SPLIT
_NBUF = 3
_LANES = 16
_COLS = _D // _LANES

_mesh = plsc.VectorSubcoreMesh(core_axis_name="c", subcore_axis_name="s")


@functools.partial(
    pl.kernel,
    mesh=_mesh,
    out_type=jax.ShapeDtypeStruct((_N, _D), jnp.float32),
    scratch_types=[
        pltpu.VMEM((_T_PER_W, _D), jnp.float32),
        pltpu.VMEM((_BATCH, _T_PER_W), jnp.int32),
        *[pltpu.VMEM((_CHUNK, _D), jnp.float32) for _ in range(_NBUF)],
        pltpu.SemaphoreType.DMA,
        *[pltpu.SemaphoreType.DMA for _ in range(2 * _NBUF)],
    ],
)
def _embed(x_hbm, table_hbm, pos_hbm, out_hbm,
           pos_v, idx_v, r0, r1, r2, psem, g0, g1, g2, w0, w1, w2):
    rows_v = (r0, r1, r2)
    gsem = (g0, g1, g2)
    wsem = (w0, w1, w2)
    wid = lax.axis_index("s") * _NC + lax.axis_index("c")
    t_base = wid * _T_PER_W

    pos_cp = pltpu.async_copy(pos_hbm.at[pl.ds(t_base, _T_PER_W)], pos_v, psem)
    for b in range(_BATCH):
        pltpu.sync_copy(
            x_hbm.at[pl.ds(b * _MAX_LEN + t_base, _T_PER_W)], idx_v.at[b])

    def chunk_base(k):
        b, c = divmod(k, _SPLIT)
        return b * _MAX_LEN + t_base + c * _CHUNK

    def fire_gather(k):
        b, c = divmod(k, _SPLIT)
        idx = idx_v.at[b, pl.ds(c * _CHUNK, _CHUNK)]
        j = k % _NBUF
        return pltpu.async_copy(table_hbm.at[idx], rows_v[j], gsem[j])

    gathers = {0: fire_gather(0)}
    writes = {}
    pos_cp.wait()
    for k in range(_NCHUNKS):
        j = k % _NBUF
        if k + 1 < _NCHUNKS:
            if k + 1 - _NBUF in writes:
                writes[k + 1 - _NBUF].wait()
            gathers[k + 1] = fire_gather(k + 1)
        gathers[k].wait()

        t_local = (k % _SPLIT) * _CHUNK
        rv = rows_v[j]

        @plsc.parallel_loop(0, _CHUNK, unroll=2)
        def _row(r):
            for col in range(_COLS):
                sl = pl.ds(col * _LANES, _LANES)
                rv[r, sl] = rv[r, sl] * _SCALE + pos_v[t_local + r, sl]

        writes[k] = pltpu.async_copy(
            rv, out_hbm.at[pl.ds(chunk_base(k), _CHUNK)], wsem[j])
    for k in range(_NCHUNKS - _NBUF, _NCHUNKS):
        writes[k].wait()


def kernel(x, table):
    xf = jnp.reshape(x, (_N,)).astype(jnp.int32)
    out = _embed(xf, table, jnp.asarray(_POS_ENC))
    return jnp.reshape(out, (_BATCH, _MAX_LEN, _D))


# speedup vs baseline: 1.4327x; 1.1108x over previous
import functools

import numpy as np
import jax
import jax.numpy as jnp
from jax import lax
from jax.experimental import pallas as pl
from jax.experimental.pallas import tpu as pltpu
from jax.experimental.pallas import tpu_sc as plsc

_D = 768
_MAX_LEN = 2048
_BATCH = 4
_SCALE = float(np.sqrt(np.float32(_D)))


def _positional_encoding() -> np.ndarray:
    pos = np.arange(_MAX_LEN)[:, np.newaxis].astype(np.float64)
    i = np.arange(_D)[np.newaxis, :].astype(np.float64)
    angle_rates = 1.0 / np.power(10000.0, 2.0 * (i // 2) / np.float32(_D))
    angle_rads = pos * angle_rates
    angle_rads[:, 0::2] = np.sin(angle_rads[:, 0::2])
    angle_rads[:, 1::2] = np.cos(angle_rads[:, 1::2])
    return angle_rads.astype(np.float32)


_POS_ENC = _positional_encoding()

_INFO = plsc.get_sparse_core_info()
_NC = _INFO.num_cores
_NS = _INFO.num_subcores
_NW = _NC * _NS
_T_PER_W = _MAX_LEN // _NW
_TCH = 16
_NCH = _T_PER_W // _TCH
_NRING = 2
_LANES = 16
_COLS = _D // _LANES

_mesh = plsc.VectorSubcoreMesh(core_axis_name="c", subcore_axis_name="s")


@functools.partial(
    pl.kernel,
    mesh=_mesh,
    out_type=jax.ShapeDtypeStruct((_BATCH, _MAX_LEN, _D), jnp.float32),
    scratch_types=[
        pltpu.VMEM((_BATCH, _T_PER_W), jnp.int32),
        *[pltpu.VMEM((_TCH, _D), jnp.float32) for _ in range(_NRING)],
        *[pltpu.VMEM((_TCH, _D), jnp.float32)
          for _ in range(_NRING * _BATCH)],
        *[pltpu.SemaphoreType.DMA for _ in range(2 * _NRING)],
        *[pltpu.SemaphoreType.DMA for _ in range(_NRING * _BATCH)],
    ],
)
def _embed(x_hbm, table_hbm, pos_hbm, out_hbm, idx_v, *refs):
    pos_v = refs[:_NRING]
    rows_v = [refs[_NRING + g * _BATCH:_NRING + (g + 1) * _BATCH]
              for g in range(_NRING)]
    psem = refs[_NRING * (1 + _BATCH):_NRING * (2 + _BATCH)]
    wsem = [refs[_NRING * (2 + _BATCH) + g * _BATCH:]
            [:_BATCH] for g in range(_NRING)]
    gsem = refs[_NRING * (2 + _BATCH) + _NRING * _BATCH:]

    wid = lax.axis_index("s") * _NC + lax.axis_index("c")
    t_base = wid * _T_PER_W

    for b in range(_BATCH):
        pltpu.sync_copy(x_hbm.at[b, pl.ds(t_base, _T_PER_W)], idx_v.at[b])

    def fire_chunk(ct):
        g = ct % _NRING
        t0 = t_base + ct * _TCH
        pos_cp = pltpu.async_copy(pos_hbm.at[pl.ds(t0, _TCH)], pos_v[g],
                                  psem[g])
        row_cps = []
        for b in range(_BATCH):
            idx = idx_v.at[b, pl.ds(ct * _TCH, _TCH)]
            row_cps.append(
                pltpu.async_copy(table_hbm.at[idx], rows_v[g][b], gsem[g]))
        return pos_cp, row_cps

    chunks = {0: fire_chunk(0)}
    writes = {}
    for ct in range(_NCH):
        g = ct % _NRING
        if ct + 1 < _NCH:
            if ct - 1 in writes:
                for w in writes[ct - 1]:
                    w.wait()
            chunks[ct + 1] = fire_chunk(ct + 1)
        pos_cp, row_cps = chunks[ct]
        pos_cp.wait()
        for cp in row_cps:
            cp.wait()

        pv = pos_v[g]
        bufs = rows_v[g]

        @plsc.parallel_loop(0, _TCH * _COLS, unroll=4)
        def _slice(i):
            r = i // _COLS
            col = i - r * _COLS
            sl = pl.ds(col * _LANES, _LANES)
            p = pv[r, sl]
            for b in range(_BATCH):
                rb = bufs[b]
                rb[r, sl] = rb[r, sl] * _SCALE + p

        t0 = t_base + ct * _TCH
        writes[ct] = [
            pltpu.async_copy(bufs[b], out_hbm.at[b, pl.ds(t0, _TCH)],
                             wsem[g][b])
            for b in range(_BATCH)
        ]
    for ct in (_NCH - 2, _NCH - 1):
        for w in writes[ct]:
            w.wait()


def kernel(x, table):
    return _embed(x.astype(jnp.int32), table, jnp.asarray(_POS_ENC))


# speedup vs baseline: 1.4399x; 1.0050x over previous
import functools

import numpy as np
import jax
import jax.numpy as jnp
from jax import lax
from jax.experimental import pallas as pl
from jax.experimental.pallas import tpu as pltpu
from jax.experimental.pallas import tpu_sc as plsc

_D = 768
_MAX_LEN = 2048
_BATCH = 4
_SCALE = float(np.sqrt(np.float32(_D)))


def _positional_encoding() -> np.ndarray:
    pos = np.arange(_MAX_LEN)[:, np.newaxis].astype(np.float64)
    i = np.arange(_D)[np.newaxis, :].astype(np.float64)
    angle_rates = 1.0 / np.power(10000.0, 2.0 * (i // 2) / np.float32(_D))
    angle_rads = pos * angle_rates
    angle_rads[:, 0::2] = np.sin(angle_rads[:, 0::2])
    angle_rads[:, 1::2] = np.cos(angle_rads[:, 1::2])
    return angle_rads.astype(np.float32)


_POS_ENC = _positional_encoding()

_INFO = plsc.get_sparse_core_info()
_NC = _INFO.num_cores
_NS = _INFO.num_subcores
_NW = _NC * _NS
_T_PER_W = _MAX_LEN // _NW
_TCH = 8
_NCH = _T_PER_W // _TCH
_NRING = 3
_LANES = 16
_COLS = _D // _LANES

_mesh = plsc.VectorSubcoreMesh(core_axis_name="c", subcore_axis_name="s")


@functools.partial(
    pl.kernel,
    mesh=_mesh,
    out_type=jax.ShapeDtypeStruct((_BATCH, _MAX_LEN, _D), jnp.float32),
    scratch_types=[
        pltpu.VMEM((_BATCH, _T_PER_W), jnp.int32),
        *[pltpu.VMEM((_TCH, _D), jnp.float32) for _ in range(_NRING)],
        *[pltpu.VMEM((_TCH, _D), jnp.float32)
          for _ in range(_NRING * _BATCH)],
        *[pltpu.SemaphoreType.DMA for _ in range(2 * _NRING)],
        *[pltpu.SemaphoreType.DMA for _ in range(_NRING * _BATCH)],
        pltpu.SemaphoreType.DMA,
    ],
)
def _embed(x_hbm, table_hbm, pos_hbm, out_hbm, idx_v, *refs):
    pos_v = refs[:_NRING]
    rows_v = [refs[_NRING + g * _BATCH:_NRING + (g + 1) * _BATCH]
              for g in range(_NRING)]
    psem = refs[_NRING * (1 + _BATCH):_NRING * (2 + _BATCH)]
    wsem = [refs[_NRING * (2 + _BATCH) + g * _BATCH:]
            [:_BATCH] for g in range(_NRING)]
    gsem = refs[_NRING * (2 + _BATCH) + _NRING * _BATCH:][:_NRING]
    isem = refs[-1]

    wid = lax.axis_index("s") * _NC + lax.axis_index("c")
    t_base = wid * _T_PER_W

    idx_cps = [
        pltpu.async_copy(x_hbm.at[b, pl.ds(t_base, _T_PER_W)],
                         idx_v.at[b], isem)
        for b in range(_BATCH)
    ]
    for cp in idx_cps:
        cp.wait()

    def fire_chunk(ct):
        g = ct % _NRING
        t0 = t_base + ct * _TCH
        pos_cp = pltpu.async_copy(pos_hbm.at[pl.ds(t0, _TCH)], pos_v[g],
                                  psem[g])
        row_cps = []
        for b in range(_BATCH):
            idx = idx_v.at[b, pl.ds(ct * _TCH, _TCH)]
            row_cps.append(
                pltpu.async_copy(table_hbm.at[idx], rows_v[g][b], gsem[g]))
        return pos_cp, row_cps

    chunks = {0: fire_chunk(0)}
    writes = {}
    for ct in range(_NCH):
        g = ct % _NRING
        if ct + 1 < _NCH:
            if ct + 1 - _NRING in writes:
                for w in writes[ct + 1 - _NRING]:
                    w.wait()
            chunks[ct + 1] = fire_chunk(ct + 1)
        pos_cp, row_cps = chunks[ct]
        pos_cp.wait()
        for cp in row_cps:
            cp.wait()

        pv = pos_v[g]
        bufs = rows_v[g]

        @plsc.parallel_loop(0, _TCH * _COLS, unroll=4)
        def _slice(i):
            r = i // _COLS
            col = i - r * _COLS
            sl = pl.ds(col * _LANES, _LANES)
            p = pv[r, sl]
            for b in range(_BATCH):
                rb = bufs[b]
                rb[r, sl] = rb[r, sl] * _SCALE + p

        t0 = t_base + ct * _TCH
        writes[ct] = [
            pltpu.async_copy(bufs[b], out_hbm.at[b, pl.ds(t0, _TCH)],
                             wsem[g][b])
            for b in range(_BATCH)
        ]
    for ct in range(_NCH - _NRING, _NCH):
        for w in writes[ct]:
            w.wait()


def kernel(x, table):
    return _embed(x.astype(jnp.int32), table, jnp.asarray(_POS_ENC))


# speedup vs baseline: 1.5050x; 1.0452x over previous
import functools

import numpy as np
import jax
import jax.numpy as jnp
from jax import lax
from jax.experimental import pallas as pl
from jax.experimental.pallas import tpu as pltpu
from jax.experimental.pallas import tpu_sc as plsc

_D = 768
_MAX_LEN = 2048
_BATCH = 4
_SCALE = float(np.sqrt(np.float32(_D)))


def _positional_encoding() -> np.ndarray:
    pos = np.arange(_MAX_LEN)[:, np.newaxis].astype(np.float64)
    i = np.arange(_D)[np.newaxis, :].astype(np.float64)
    angle_rates = 1.0 / np.power(10000.0, 2.0 * (i // 2) / np.float32(_D))
    angle_rads = pos * angle_rates
    angle_rads[:, 0::2] = np.sin(angle_rads[:, 0::2])
    angle_rads[:, 1::2] = np.cos(angle_rads[:, 1::2])
    return angle_rads.astype(np.float32)


_POS_ENC = _positional_encoding()

_INFO = plsc.get_sparse_core_info()
_NC = _INFO.num_cores
_NS = _INFO.num_subcores
_NW = _NC * _NS
_T_PER_W = _MAX_LEN // _NW
_TCH = 8
_NCH = _T_PER_W // _TCH
_NRING = 4
_PREF = 2
_LANES = 16
_COLS = _D // _LANES

_mesh = plsc.VectorSubcoreMesh(core_axis_name="c", subcore_axis_name="s")


@functools.partial(
    pl.kernel,
    mesh=_mesh,
    out_type=jax.ShapeDtypeStruct((_BATCH, _MAX_LEN, _D), jnp.float32),
    scratch_types=[
        pltpu.VMEM((_BATCH, _T_PER_W), jnp.int32),
        *[pltpu.VMEM((_TCH, _D), jnp.float32) for _ in range(_NRING)],
        *[pltpu.VMEM((_TCH, _D), jnp.float32)
          for _ in range(_NRING * _BATCH)],
        *[pltpu.SemaphoreType.DMA for _ in range(2 * _NRING)],
        *[pltpu.SemaphoreType.DMA for _ in range(_NRING * _BATCH)],
        pltpu.SemaphoreType.DMA,
    ],
)
def _embed(x_hbm, table_hbm, pos_hbm, out_hbm, idx_v, *refs):
    pos_v = refs[:_NRING]
    rows_v = [refs[_NRING + g * _BATCH:_NRING + (g + 1) * _BATCH]
              for g in range(_NRING)]
    psem = refs[_NRING * (1 + _BATCH):_NRING * (2 + _BATCH)]
    wsem = [refs[_NRING * (2 + _BATCH) + g * _BATCH:]
            [:_BATCH] for g in range(_NRING)]
    gsem = refs[_NRING * (2 + _BATCH) + _NRING * _BATCH:][:_NRING]
    isem = refs[-1]

    wid = lax.axis_index("s") * _NC + lax.axis_index("c")
    t_base = wid * _T_PER_W

    idx_cps = [
        pltpu.async_copy(x_hbm.at[b, pl.ds(t_base, _T_PER_W)],
                         idx_v.at[b], isem)
        for b in range(_BATCH)
    ]
    for cp in idx_cps:
        cp.wait()

    def fire_chunk(ct):
        g = ct % _NRING
        t0 = t_base + ct * _TCH
        pos_cp = pltpu.async_copy(pos_hbm.at[pl.ds(t0, _TCH)], pos_v[g],
                                  psem[g])
        row_cps = []
        for b in range(_BATCH):
            idx = idx_v.at[b, pl.ds(ct * _TCH, _TCH)]
            row_cps.append(
                pltpu.async_copy(table_hbm.at[idx], rows_v[g][b], gsem[g]))
        return pos_cp, row_cps

    chunks = {k: fire_chunk(k) for k in range(_PREF)}
    writes = {}
    for ct in range(_NCH):
        g = ct % _NRING
        if ct + _PREF < _NCH:
            if ct + _PREF - _NRING in writes:
                for w in writes[ct + _PREF - _NRING]:
                    w.wait()
            chunks[ct + _PREF] = fire_chunk(ct + _PREF)
        pos_cp, row_cps = chunks[ct]
        pos_cp.wait()
        for cp in row_cps:
            cp.wait()

        pv = pos_v[g]
        bufs = rows_v[g]

        @plsc.parallel_loop(0, _TCH * _COLS, unroll=4)
        def _slice(i):
            r = i // _COLS
            col = i - r * _COLS
            sl = pl.ds(col * _LANES, _LANES)
            p = pv[r, sl]
            for b in range(_BATCH):
                rb = bufs[b]
                rb[r, sl] = rb[r, sl] * _SCALE + p

        t0 = t_base + ct * _TCH
        writes[ct] = [
            pltpu.async_copy(bufs[b], out_hbm.at[b, pl.ds(t0, _TCH)],
                             wsem[g][b])
            for b in range(_BATCH)
        ]
    for ct in range(_NCH - _NRING, _NCH):
        for w in writes[ct]:
            w.wait()


def kernel(x, table):
    return _embed(x.astype(jnp.int32), table, jnp.asarray(_POS_ENC))
